# baseline (speedup 1.0000x reference)
import jax
import jax.numpy as jnp
from jax.experimental import pallas as pl

N_NODES = 10000


def _bn(x, g, b):
    m = jnp.mean(x, axis=0)
    v = jnp.var(x, axis=0)
    return (x - m) / jnp.sqrt(v + 1e-5) * g + b


def _gine(x, ei, ea, p):
    e = ea @ p['edge_W'] + p['edge_b']
    msg = jax.nn.relu(x[ei[0]] + e)
    aggr = jax.ops.segment_sum(msg, ei[1], num_segments=x.shape[0])
    h = (1.0 + p['eps']) * x + aggr
    return jax.nn.relu(h @ p['W1'] + p['b1']) @ p['W2'] + p['b2']


def _scatter_mean(data, ids, size):
    s = jax.ops.segment_sum(data, ids, num_segments=size)
    c = jax.ops.segment_sum(jnp.ones((data.shape[0], 1), data.dtype), ids, num_segments=size)
    return s / jnp.clip(c, 1.0, None)


def _relu_kernel(a_ref, o_ref):
    o_ref[...] = jnp.maximum(a_ref[...], 0.0)


def kernel(h_flat, intra_ei, ea_flat, valid, node_ids, N_total, edge_index, edge_attr,
           sub_batch, S, k, root_flat_idx, is_root, params):
    F = h_flat.shape[0]
    h_skip = h_flat @ params['skip_W'] + params['skip_b']
    lp = params['local']
    h1 = _bn(_gine(h_flat, intra_ei, ea_flat, lp), lp['bn_g'], lp['bn_b'])
    x_sum = _scatter_mean(h_flat, node_ids, N_NODES)
    gp = params['global']
    h2 = _bn(_gine(x_sum, edge_index, edge_attr, gp), gp['bn_g'], gp['bn_b'])[node_ids]
    root_ids = node_ids[root_flat_idx]
    x_vv_can = _scatter_mean(h_flat[root_flat_idx], root_ids, N_NODES)
    x_vv = x_vv_can[node_ids] @ params['vv_W'] + params['vv_b']
    x_kk = h_flat[root_flat_idx[sub_batch]] @ params['kk_W'] + params['kk_b']
    pre = h_skip + h1 + h2 + x_vv + x_kk
    out = pl.pallas_call(
        _relu_kernel,
        out_shape=jax.ShapeDtypeStruct(pre.shape, pre.dtype),
        grid=(F // 8192,),
        in_specs=[pl.BlockSpec((8192, 128), lambda i: (i, 0))],
        out_specs=pl.BlockSpec((8192, 128), lambda i: (i, 0)),
    )(pre)
    return out


# speedup vs baseline: 2.2612x; 2.2612x over previous
import functools

import jax
import jax.numpy as jnp
from jax import lax
from jax.experimental import pallas as pl
from jax.experimental.pallas import tpu as pltpu
from jax.experimental.pallas import tpu_sc as plsc

F_TOTAL = 131072
H = 128
N_NODES = 10000
NC = 2
NS = 16
NW = NC * NS
CH = 128
HALF = 5120
TROWS = 5632
TRASH = HALF

_mesh = plsc.VectorSubcoreMesh(core_axis_name="c", subcore_axis_name="s",
                               num_cores=NC, num_subcores=NS)


def _fill_vmem(ref, rows, width, value):
    v = jnp.full((16,), value, jnp.float32)

    def row(i, _):
        for j in range(width // 16):
            ref[i, pl.ds(j * 16, 16)] = v
        return 0

    lax.fori_loop(0, rows, row, 0)


def _zero_stripe(table, sid, zsrc):
    base = sid * (TROWS // NS)
    pltpu.sync_copy(zsrc, table.at[pl.ds(base, CH)])
    pltpu.sync_copy(zsrc, table.at[pl.ds(base + CH, CH)])
    pltpu.sync_copy(zsrc.at[pl.ds(0, 96)], table.at[pl.ds(base + 2 * CH, 96)])


def _scatter_mean_body(nrows, data_hbm, ids_hbm, out_sums, out_counts,
                       hbuf, idxbuf, onesbuf, ssum, scnt):
    cid = lax.axis_index("c")
    sid = lax.axis_index("s")

    _fill_vmem(hbuf, CH, H, 0.0)
    _fill_vmem(onesbuf, CH, H, 1.0)
    _zero_stripe(ssum, sid, hbuf)
    _zero_stripe(scnt, sid, hbuf)
    plsc.subcore_barrier()

    per_w = nrows // NS
    nch = per_w // CH
    lo = cid * HALF

    def chunk(i, _):
        base = sid * per_w + i * CH
        pltpu.sync_copy(data_hbm.at[pl.ds(base, CH)], hbuf)
        pltpu.sync_copy(ids_hbm.at[pl.ds(base, CH)], idxbuf.at[0])
        for j in range(CH // 16):
            v = idxbuf[0, pl.ds(j * 16, 16)] - lo
            oob = (v < 0) | (v >= HALF)
            idxbuf[0, pl.ds(j * 16, 16)] = jnp.where(oob, TRASH, v)
        pltpu.sync_copy(hbuf, ssum.at[idxbuf.at[0]], add=True)
        pltpu.sync_copy(onesbuf, scnt.at[idxbuf.at[0]], add=True)
        return 0

    lax.fori_loop(0, nch, chunk, 0)
    plsc.subcore_barrier()

    fps = HALF // NS
    pltpu.sync_copy(ssum.at[pl.ds(sid * fps, fps)],
                    out_sums.at[pl.ds(cid * HALF + sid * fps, fps)])
    pltpu.sync_copy(scnt.at[pl.ds(sid * fps, fps)],
                    out_counts.at[pl.ds(cid * HALF + sid * fps, fps)])


def _scatter_mean_sc(data, ids):
    nrows = data.shape[0]
    body = functools.partial(_scatter_mean_body, nrows)
    sums, counts = pl.kernel(
        body,
        out_type=[
            jax.ShapeDtypeStruct((NC * HALF, H), jnp.float32),
            jax.ShapeDtypeStruct((NC * HALF, H), jnp.float32),
        ],
        mesh=_mesh,
        scratch_types=[
            pltpu.VMEM((CH, H), jnp.float32),
            pltpu.VMEM((1, CH), jnp.int32),
            pltpu.VMEM((CH, H), jnp.float32),
            pltpu.VMEM_SHARED((TROWS, H), jnp.float32),
            pltpu.VMEM_SHARED((TROWS, H), jnp.float32),
        ],
    )(data, ids)
    return sums[:N_NODES] / jnp.clip(counts[:N_NODES, 0:1], 1.0, None)


def _bn(x, g, b):
    m = jnp.mean(x, axis=0)
    v = jnp.var(x, axis=0)
    return (x - m) / jnp.sqrt(v + 1e-5) * g + b


def _gine(x, ei, ea, p):
    e = ea @ p['edge_W'] + p['edge_b']
    msg = jax.nn.relu(x[ei[0]] + e)
    aggr = jax.ops.segment_sum(msg, ei[1], num_segments=x.shape[0])
    h = (1.0 + p['eps']) * x + aggr
    return jax.nn.relu(h @ p['W1'] + p['b1']) @ p['W2'] + p['b2']


def kernel(h_flat, intra_ei, ea_flat, valid, node_ids, N_total, edge_index, edge_attr,
           sub_batch, S, k, root_flat_idx, is_root, params):
    h_skip = h_flat @ params['skip_W'] + params['skip_b']
    lp = params['local']
    h1 = _bn(_gine(h_flat, intra_ei, ea_flat, lp), lp['bn_g'], lp['bn_b'])
    x_sum = _scatter_mean_sc(h_flat, node_ids)
    gp = params['global']
    h2 = _bn(_gine(x_sum, edge_index, edge_attr, gp), gp['bn_g'], gp['bn_b'])[node_ids]
    root_ids = node_ids[root_flat_idx]
    roots_h = h_flat[root_flat_idx]
    x_vv_can = _scatter_mean_sc(roots_h, root_ids)
    x_vv = x_vv_can[node_ids] @ params['vv_W'] + params['vv_b']
    x_kk = roots_h[sub_batch] @ params['kk_W'] + params['kk_b']
    return jax.nn.relu(h_skip + h1 + h2 + x_vv + x_kk)


# speedup vs baseline: 3.0214x; 1.3362x over previous
import functools

import jax
import jax.numpy as jnp
from jax import lax
from jax.experimental import pallas as pl
from jax.experimental.pallas import tpu as pltpu
from jax.experimental.pallas import tpu_sc as plsc

F_TOTAL = 131072
H = 128
N_NODES = 10000
NC = 2
NS = 16
NW = NC * NS
CH = 128
HALF = 5120
TROWS = 5632
TRASH = HALF

_mesh = plsc.VectorSubcoreMesh(core_axis_name="c", subcore_axis_name="s",
                               num_cores=NC, num_subcores=NS)


def _fill_vmem(ref, rows, width, value):
    v = jnp.full((16,), value, jnp.float32)

    def row(i, _):
        for j in range(width // 16):
            ref[i, pl.ds(j * 16, 16)] = v
        return 0

    lax.fori_loop(0, rows, row, 0)


def _zero_stripe(table, sid, zsrc):
    base = sid * (TROWS // NS)
    pltpu.sync_copy(zsrc, table.at[pl.ds(base, CH)])
    pltpu.sync_copy(zsrc, table.at[pl.ds(base + CH, CH)])
    pltpu.sync_copy(zsrc.at[pl.ds(0, 96)], table.at[pl.ds(base + 2 * CH, 96)])


def _scatter_mean_body(nrows, data_hbm, ids_hbm, out_sums, out_counts,
                       hbuf, idxbuf, onesbuf, ssum, scnt):
    cid = lax.axis_index("c")
    sid = lax.axis_index("s")

    _fill_vmem(hbuf, CH, H, 0.0)
    _fill_vmem(onesbuf, CH, H, 1.0)
    _zero_stripe(ssum, sid, hbuf)
    _zero_stripe(scnt, sid, hbuf)
    plsc.subcore_barrier()

    per_w = nrows // NS
    nch = per_w // CH
    lo = cid * HALF

    def chunk(i, _):
        base = sid * per_w + i * CH
        pltpu.sync_copy(data_hbm.at[pl.ds(base, CH)], hbuf)
        pltpu.sync_copy(ids_hbm.at[pl.ds(base, CH)], idxbuf.at[0])
        for j in range(CH // 16):
            v = idxbuf[0, pl.ds(j * 16, 16)] - lo
            oob = (v < 0) | (v >= HALF)
            idxbuf[0, pl.ds(j * 16, 16)] = jnp.where(oob, TRASH, v)
        pltpu.sync_copy(hbuf, ssum.at[idxbuf.at[0]], add=True)
        pltpu.sync_copy(onesbuf, scnt.at[idxbuf.at[0]], add=True)
        return 0

    lax.fori_loop(0, nch, chunk, 0)
    plsc.subcore_barrier()

    fps = HALF // NS
    pltpu.sync_copy(ssum.at[pl.ds(sid * fps, fps)],
                    out_sums.at[pl.ds(cid * HALF + sid * fps, fps)])
    pltpu.sync_copy(scnt.at[pl.ds(sid * fps, fps)],
                    out_counts.at[pl.ds(cid * HALF + sid * fps, fps)])


def _scatter_mean_sc(data, ids):
    nrows = data.shape[0]
    body = functools.partial(_scatter_mean_body, nrows)
    sums, counts = pl.kernel(
        body,
        out_type=[
            jax.ShapeDtypeStruct((NC * HALF, H), jnp.float32),
            jax.ShapeDtypeStruct((NC * HALF, H), jnp.float32),
        ],
        mesh=_mesh,
        scratch_types=[
            pltpu.VMEM((CH, H), jnp.float32),
            pltpu.VMEM((1, CH), jnp.int32),
            pltpu.VMEM((CH, H), jnp.float32),
            pltpu.VMEM_SHARED((TROWS, H), jnp.float32),
            pltpu.VMEM_SHARED((TROWS, H), jnp.float32),
        ],
    )(data, ids)
    return sums[:N_NODES] / jnp.clip(counts[:N_NODES, 0:1], 1.0, None)


E_GLOBAL = 320000
GT_ROWS = 10240
_G_CHUNKS = E_GLOBAL // CH
_G_FULL = (_G_CHUNKS // NW) * NW
_G_EXTRA = _G_CHUNKS - _G_FULL


def _gaggr_body(xsum_hbm, srcs_hbm, dsts_hbm, eg_hbm, out_hbm,
                hbuf, ebuf, sidx, didx, stable):
    cid = lax.axis_index("c")
    sid = lax.axis_index("s")
    wid = sid * NC + cid

    _fill_vmem(hbuf, CH, H, 0.0)
    zb = sid * (GT_ROWS // NS)
    for rep in range(GT_ROWS // NS // CH):
        pltpu.sync_copy(hbuf, stable.at[pl.ds(zb + rep * CH, CH)])
    plsc.subcore_barrier()

    def do_chunk(c):
        e0 = c * CH
        pltpu.sync_copy(srcs_hbm.at[pl.ds(e0, CH)], sidx.at[0])
        pltpu.sync_copy(eg_hbm.at[pl.ds(e0, CH)], ebuf)
        pltpu.sync_copy(xsum_hbm.at[sidx.at[0]], hbuf)

        def row(r, _):
            for j in range(H // 16):
                v = hbuf[r, pl.ds(j * 16, 16)] + ebuf[r, pl.ds(j * 16, 16)]
                hbuf[r, pl.ds(j * 16, 16)] = jnp.maximum(v, 0.0)
            return 0

        lax.fori_loop(0, CH, row, 0)
        pltpu.sync_copy(dsts_hbm.at[pl.ds(e0, CH)], didx.at[0])
        pltpu.sync_copy(hbuf, stable.at[didx.at[0]], add=True)

    def it(i, _):
        do_chunk(wid + i * NW)
        return 0

    lax.fori_loop(0, _G_FULL // NW, it, 0)

    @pl.when(wid < _G_EXTRA)
    def _():
        do_chunk(_G_FULL + wid)

    plsc.subcore_barrier()
    for rep in range(GT_ROWS // NS // CH):
        pltpu.sync_copy(stable.at[pl.ds(zb + rep * CH, CH)],
                        out_hbm.at[pl.ds(cid * GT_ROWS + zb + rep * CH, CH)])


def _global_aggr_sc(x_sum, edge_index, e_g):
    partials = pl.kernel(
        _gaggr_body,
        out_type=jax.ShapeDtypeStruct((NC * GT_ROWS, H), jnp.float32),
        mesh=_mesh,
        scratch_types=[
            pltpu.VMEM((CH, H), jnp.float32),
            pltpu.VMEM((CH, H), jnp.float32),
            pltpu.VMEM((1, CH), jnp.int32),
            pltpu.VMEM((1, CH), jnp.int32),
            pltpu.VMEM_SHARED((GT_ROWS, H), jnp.float32),
        ],
    )(x_sum, edge_index[0], edge_index[1], e_g)
    return partials[:N_NODES] + partials[GT_ROWS:GT_ROWS + N_NODES]


def _bn(x, g, b):
    m = jnp.mean(x, axis=0)
    v = jnp.var(x, axis=0)
    return (x - m) / jnp.sqrt(v + 1e-5) * g + b


def _gine(x, ei, ea, p):
    e = ea @ p['edge_W'] + p['edge_b']
    msg = jax.nn.relu(x[ei[0]] + e)
    aggr = jax.ops.segment_sum(msg, ei[1], num_segments=x.shape[0])
    h = (1.0 + p['eps']) * x + aggr
    return jax.nn.relu(h @ p['W1'] + p['b1']) @ p['W2'] + p['b2']


def kernel(h_flat, intra_ei, ea_flat, valid, node_ids, N_total, edge_index, edge_attr,
           sub_batch, S, k, root_flat_idx, is_root, params):
    h_skip = h_flat @ params['skip_W'] + params['skip_b']
    lp = params['local']
    h1 = _bn(_gine(h_flat, intra_ei, ea_flat, lp), lp['bn_g'], lp['bn_b'])
    x_sum = _scatter_mean_sc(h_flat, node_ids)
    gp = params['global']
    e_g = edge_attr @ gp['edge_W'] + gp['edge_b']
    aggr_g = _global_aggr_sc(x_sum, edge_index, e_g)
    t_g = (1.0 + gp['eps']) * x_sum + aggr_g
    g2 = jax.nn.relu(t_g @ gp['W1'] + gp['b1']) @ gp['W2'] + gp['b2']
    h2 = _bn(g2, gp['bn_g'], gp['bn_b'])[node_ids]
    root_ids = node_ids[root_flat_idx]
    roots_h = h_flat[root_flat_idx]
    x_vv_can = _scatter_mean_sc(roots_h, root_ids)
    x_vv = x_vv_can[node_ids] @ params['vv_W'] + params['vv_b']
    x_kk = roots_h[sub_batch] @ params['kk_W'] + params['kk_b']
    return jax.nn.relu(h_skip + h1 + h2 + x_vv + x_kk)
